# baseline (device time: 203055 ns/iter reference)
import jax
import jax.numpy as jnp
from jax import lax
from jax.experimental import pallas as pl
from jax.experimental.pallas import tpu as pltpu

N_DEV = 16


def _silu(y):
    return y * jax.nn.sigmoid(y)


def kernel(x, w_mat):
    m_per, k = x.shape
    _, n_per = w_mat.shape

    def body(x_ref, w_ref, out_ref, comm_ref, send_sems, recv_sems):
        my = lax.axis_index("i")
        left = lax.rem(my - 1 + N_DEV, N_DEV)
        right = lax.rem(my + 1, N_DEV)

        barrier_sem = pltpu.get_barrier_semaphore()
        for nbr in (left, right):
            pl.semaphore_signal(
                barrier_sem, inc=1,
                device_id=(nbr,), device_id_type=pl.DeviceIdType.MESH,
            )
        pl.semaphore_wait(barrier_sem, 2)

        comm_ref[0, :, :] = x_ref[:, :]

        for h in range(N_DEV - 1):
            send_slot = h % 2
            recv_slot = (h + 1) % 2
            rdma = pltpu.make_async_remote_copy(
                src_ref=comm_ref.at[send_slot],
                dst_ref=comm_ref.at[recv_slot],
                send_sem=send_sems.at[send_slot],
                recv_sem=recv_sems.at[recv_slot],
                device_id=(right,),
                device_id_type=pl.DeviceIdType.MESH,
            )
            rdma.start()
            origin = lax.rem(my - h + N_DEV, N_DEV)
            y = jnp.dot(comm_ref[send_slot, :, :], w_ref[:, :],
                        preferred_element_type=jnp.float32)
            out_ref[pl.ds(origin * m_per, m_per), :] = _silu(y)
            rdma.wait()

        origin = lax.rem(my + 1, N_DEV)
        y = jnp.dot(comm_ref[1, :, :], w_ref[:, :],
                    preferred_element_type=jnp.float32)
        out_ref[pl.ds(origin * m_per, m_per), :] = _silu(y)

    return pl.pallas_call(
        body,
        out_shape=jax.ShapeDtypeStruct((N_DEV * m_per, n_per), jnp.float32),
        in_specs=[
            pl.BlockSpec(memory_space=pltpu.VMEM),
            pl.BlockSpec(memory_space=pltpu.VMEM),
        ],
        out_specs=pl.BlockSpec(memory_space=pltpu.VMEM),
        scratch_shapes=[
            pltpu.VMEM((2, m_per, k), jnp.float32),
            pltpu.SemaphoreType.DMA((2,)),
            pltpu.SemaphoreType.DMA((2,)),
        ],
        compiler_params=pltpu.CompilerParams(collective_id=0),
    )(x, w_mat)


# device time: 124226 ns/iter; 1.6346x vs baseline; 1.6346x over previous
import jax
import jax.numpy as jnp
from jax import lax
from jax.experimental import pallas as pl
from jax.experimental.pallas import tpu as pltpu

N_DEV = 16
H_P = 8
H_M = 7


def _silu(y):
    return y * jax.nn.sigmoid(y)


def kernel(x, w_mat):
    m_per, k = x.shape
    _, n_per = w_mat.shape

    def body(x_ref, w_ref, out_ref,
             comm_p, comm_m, send_p, recv_p, send_m, recv_m):
        my = lax.axis_index("i")
        left = lax.rem(my - 1 + N_DEV, N_DEV)
        right = lax.rem(my + 1, N_DEV)

        barrier_sem = pltpu.get_barrier_semaphore()
        for nbr in (left, right):
            pl.semaphore_signal(
                barrier_sem, inc=1,
                device_id=(nbr,), device_id_type=pl.DeviceIdType.MESH,
            )
        pl.semaphore_wait(barrier_sem, 2)

        comm_p[0, :, :] = x_ref[:, :]
        comm_m[0, :, :] = x_ref[:, :]

        def gemm_block(src_ref, slot, origin):
            y = jnp.dot(src_ref[slot, :, :], w_ref[:, :],
                        preferred_element_type=jnp.float32)
            out_ref[pl.ds(origin * m_per, m_per), :] = _silu(y)

        for h in range(H_P):
            s, r = h % 2, (h + 1) % 2
            rdma_p = pltpu.make_async_remote_copy(
                src_ref=comm_p.at[s], dst_ref=comm_p.at[r],
                send_sem=send_p.at[s], recv_sem=recv_p.at[r],
                device_id=(right,), device_id_type=pl.DeviceIdType.MESH,
            )
            rdma_p.start()
            if h < H_M:
                rdma_m = pltpu.make_async_remote_copy(
                    src_ref=comm_m.at[s], dst_ref=comm_m.at[r],
                    send_sem=send_m.at[s], recv_sem=recv_m.at[r],
                    device_id=(left,), device_id_type=pl.DeviceIdType.MESH,
                )
                rdma_m.start()
            if h == 0:
                gemm_block(comm_p, 0, my)
            else:
                gemm_block(comm_p, s, lax.rem(my - h + N_DEV, N_DEV))
                gemm_block(comm_m, s, lax.rem(my + h, N_DEV))
            rdma_p.wait()
            if h < H_M:
                rdma_m.wait()

        gemm_block(comm_p, H_P % 2, lax.rem(my - H_P + N_DEV, N_DEV))
        gemm_block(comm_m, H_M % 2, lax.rem(my + H_M, N_DEV))

    return pl.pallas_call(
        body,
        out_shape=jax.ShapeDtypeStruct((N_DEV * m_per, n_per), jnp.float32),
        in_specs=[
            pl.BlockSpec(memory_space=pltpu.VMEM),
            pl.BlockSpec(memory_space=pltpu.VMEM),
        ],
        out_specs=pl.BlockSpec(memory_space=pltpu.VMEM),
        scratch_shapes=[
            pltpu.VMEM((2, m_per, k), jnp.float32),
            pltpu.VMEM((2, m_per, k), jnp.float32),
            pltpu.SemaphoreType.DMA((2,)),
            pltpu.SemaphoreType.DMA((2,)),
            pltpu.SemaphoreType.DMA((2,)),
            pltpu.SemaphoreType.DMA((2,)),
        ],
        compiler_params=pltpu.CompilerParams(collective_id=0),
    )(x, w_mat)


# device time: 116691 ns/iter; 1.7401x vs baseline; 1.0646x over previous
import jax
import jax.numpy as jnp
from jax import lax
from jax.experimental import pallas as pl
from jax.experimental.pallas import tpu as pltpu

N_DEV = 16
H_P = 8
H_M = 7

CYCLE = [0, 1, 5, 9, 13, 14, 10, 6, 2, 3, 7, 11, 15, 12, 8, 4]
INV = [0] * N_DEV
for _p, _d in enumerate(CYCLE):
    INV[_d] = _p


def _silu(y):
    return y * jax.nn.sigmoid(y)


def kernel(x, w_mat):
    m_per, k = x.shape
    _, n_per = w_mat.shape

    my = lax.axis_index("i")
    cyc = jnp.asarray(CYCLE, jnp.int32)
    inv = jnp.asarray(INV, jnp.int32)
    pos = inv[my]
    left = cyc[lax.rem(pos - 1 + N_DEV, N_DEV)]
    right = cyc[lax.rem(pos + 1, N_DEV)]
    origins_p = cyc[lax.rem(pos - 1 - jnp.arange(H_P, dtype=jnp.int32) + 2 * N_DEV, N_DEV)]
    origins_m = cyc[lax.rem(pos + 1 + jnp.arange(H_M, dtype=jnp.int32), N_DEV)]
    meta = jnp.concatenate(
        [jnp.stack([my.astype(jnp.int32), left, right]), origins_p, origins_m]
    )

    def body(meta_ref, x_ref, w_ref, out_ref,
             comm_p, comm_m, send_p, recv_p, send_m, recv_m):
        left_id = meta_ref[1]
        right_id = meta_ref[2]

        barrier_sem = pltpu.get_barrier_semaphore()
        for nbr in (left_id, right_id):
            pl.semaphore_signal(
                barrier_sem, inc=1,
                device_id=(nbr,), device_id_type=pl.DeviceIdType.MESH,
            )
        pl.semaphore_wait(barrier_sem, 2)

        comm_p[0, :, :] = x_ref[:, :]
        comm_m[0, :, :] = x_ref[:, :]

        def gemm_block(src_ref, slot, origin):
            y = jnp.dot(src_ref[slot, :, :], w_ref[:, :],
                        preferred_element_type=jnp.float32)
            out_ref[pl.ds(origin * m_per, m_per), :] = _silu(y)

        for h in range(H_P):
            s, r = h % 2, (h + 1) % 2
            rdma_p = pltpu.make_async_remote_copy(
                src_ref=comm_p.at[s], dst_ref=comm_p.at[r],
                send_sem=send_p.at[s], recv_sem=recv_p.at[r],
                device_id=(right_id,), device_id_type=pl.DeviceIdType.MESH,
            )
            rdma_p.start()
            if h < H_M:
                rdma_m = pltpu.make_async_remote_copy(
                    src_ref=comm_m.at[s], dst_ref=comm_m.at[r],
                    send_sem=send_m.at[s], recv_sem=recv_m.at[r],
                    device_id=(left_id,), device_id_type=pl.DeviceIdType.MESH,
                )
                rdma_m.start()
            if h == 0:
                gemm_block(comm_p, 0, meta_ref[0])
            else:
                gemm_block(comm_p, s, meta_ref[3 + (h - 1)])
                gemm_block(comm_m, s, meta_ref[3 + H_P + (h - 1)])
            rdma_p.wait()
            if h < H_M:
                rdma_m.wait()

        gemm_block(comm_p, H_P % 2, meta_ref[3 + H_P - 1])
        gemm_block(comm_m, H_M % 2, meta_ref[3 + H_P + H_M - 1])

    return pl.pallas_call(
        body,
        out_shape=jax.ShapeDtypeStruct((N_DEV * m_per, n_per), jnp.float32),
        in_specs=[
            pl.BlockSpec(memory_space=pltpu.SMEM),
            pl.BlockSpec(memory_space=pltpu.VMEM),
            pl.BlockSpec(memory_space=pltpu.VMEM),
        ],
        out_specs=pl.BlockSpec(memory_space=pltpu.VMEM),
        scratch_shapes=[
            pltpu.VMEM((2, m_per, k), jnp.float32),
            pltpu.VMEM((2, m_per, k), jnp.float32),
            pltpu.SemaphoreType.DMA((2,)),
            pltpu.SemaphoreType.DMA((2,)),
            pltpu.SemaphoreType.DMA((2,)),
            pltpu.SemaphoreType.DMA((2,)),
        ],
        compiler_params=pltpu.CompilerParams(collective_id=0),
    )(meta, x, w_mat)


# device time: 97796 ns/iter; 2.0763x vs baseline; 1.1932x over previous
import jax
import jax.numpy as jnp
from jax import lax
from jax.experimental import pallas as pl
from jax.experimental.pallas import tpu as pltpu

N_DEV = 16
NSLOT = 15

CYCLE = [0, 1, 5, 9, 13, 14, 10, 6, 2, 3, 7, 11, 15, 12, 8, 4]
INV = [0] * N_DEV
for _p, _d in enumerate(CYCLE):
    INV[_d] = _p


def _silu(y):
    return y * jax.nn.sigmoid(y)


def kernel(x, w_mat):
    m_per, kdim = x.shape
    _, n_per = w_mat.shape
    half = m_per // 2

    my = lax.axis_index("i")
    cyc = jnp.asarray(CYCLE, jnp.int32)
    inv = jnp.asarray(INV, jnp.int32)
    pos = inv[my]
    left = cyc[(pos - 1) % N_DEV]
    right = cyc[(pos + 1) % N_DEV]
    kk = jnp.arange(NSLOT, dtype=jnp.int32)
    q = kk // 2 + 1
    origin_p = cyc[(pos - q) % N_DEV]
    origin_m = cyc[(pos + q) % N_DEV]
    row_p = origin_p * m_per + jnp.where(kk % 2 == 0, 0, half)
    row_m = origin_m * m_per + jnp.where(kk % 2 == 0, half, 0)
    meta = jnp.concatenate(
        [jnp.stack([my.astype(jnp.int32) * m_per, left, right]), row_p, row_m]
    )

    def body(meta_ref, x_ref, w_ref, out_ref,
             bufp, bufm, send_p, recv_p, send_m, recv_m):
        left_id = meta_ref[1]
        right_id = meta_ref[2]

        barrier_sem = pltpu.get_barrier_semaphore()
        for nbr in (left_id, right_id):
            pl.semaphore_signal(
                barrier_sem, inc=1,
                device_id=(nbr,), device_id_type=pl.DeviceIdType.MESH,
            )
        pl.semaphore_wait(barrier_sem, 2)

        def mk(buf, send_sems, recv_sems, slot, src_ref, dev):
            return pltpu.make_async_remote_copy(
                src_ref=src_ref,
                dst_ref=buf.at[slot],
                send_sem=send_sems.at[slot],
                recv_sem=recv_sems.at[slot],
                device_id=(dev,),
                device_id_type=pl.DeviceIdType.MESH,
            )

        x_a = x_ref.at[pl.ds(0, half), :]
        x_b = x_ref.at[pl.ds(half, half), :]
        descs_p = {
            0: mk(bufp, send_p, recv_p, 0, x_a, right_id),
            1: mk(bufp, send_p, recv_p, 1, x_b, right_id),
        }
        descs_m = {
            0: mk(bufm, send_m, recv_m, 0, x_b, left_id),
            1: mk(bufm, send_m, recv_m, 1, x_a, left_id),
        }
        descs_p[0].start()
        descs_m[0].start()
        descs_p[1].start()
        descs_m[1].start()

        y = jnp.dot(x_ref[:, :], w_ref[:, :],
                    preferred_element_type=jnp.float32)
        out_ref[pl.ds(meta_ref[0], m_per), :] = _silu(y)

        for h in range(NSLOT):
            descs_p[h].wait_recv()
            if h <= NSLOT - 3:
                d = mk(bufp, send_p, recv_p, h + 2, bufp.at[h], right_id)
                descs_p[h + 2] = d
                d.start()
            descs_m[h].wait_recv()
            if h <= NSLOT - 3:
                d = mk(bufm, send_m, recv_m, h + 2, bufm.at[h], left_id)
                descs_m[h + 2] = d
                d.start()
            yp = jnp.dot(bufp[h, :, :], w_ref[:, :],
                         preferred_element_type=jnp.float32)
            out_ref[pl.ds(meta_ref[3 + h], half), :] = _silu(yp)
            ym = jnp.dot(bufm[h, :, :], w_ref[:, :],
                         preferred_element_type=jnp.float32)
            out_ref[pl.ds(meta_ref[3 + NSLOT + h], half), :] = _silu(ym)

        for h in range(NSLOT):
            descs_p[h].wait_send()
            descs_m[h].wait_send()

    return pl.pallas_call(
        body,
        out_shape=jax.ShapeDtypeStruct((N_DEV * m_per, n_per), jnp.float32),
        in_specs=[
            pl.BlockSpec(memory_space=pltpu.SMEM),
            pl.BlockSpec(memory_space=pltpu.VMEM),
            pl.BlockSpec(memory_space=pltpu.VMEM),
        ],
        out_specs=pl.BlockSpec(memory_space=pltpu.VMEM),
        scratch_shapes=[
            pltpu.VMEM((NSLOT, half, kdim), jnp.float32),
            pltpu.VMEM((NSLOT, half, kdim), jnp.float32),
            pltpu.SemaphoreType.DMA((NSLOT,)),
            pltpu.SemaphoreType.DMA((NSLOT,)),
            pltpu.SemaphoreType.DMA((NSLOT,)),
            pltpu.SemaphoreType.DMA((NSLOT,)),
        ],
        compiler_params=pltpu.CompilerParams(collective_id=0),
    )(meta, x, w_mat)


# device time: 94917 ns/iter; 2.1393x vs baseline; 1.0303x over previous
import jax
import jax.numpy as jnp
import numpy as np
from jax import lax
from jax.experimental import pallas as pl
from jax.experimental.pallas import tpu as pltpu

N_DEV = 16
NSLOT = 15

CYCLE = [0, 1, 5, 9, 13, 14, 10, 6, 2, 3, 7, 11, 15, 12, 8, 4]
INV = [0] * N_DEV
for _p, _d in enumerate(CYCLE):
    INV[_d] = _p


def _meta_table(m_per: int) -> np.ndarray:
    half = m_per // 2
    rows = []
    for d in range(N_DEV):
        pos = INV[d]
        left = CYCLE[(pos - 1) % N_DEV]
        right = CYCLE[(pos + 1) % N_DEV]
        row_p, row_m = [], []
        for k in range(NSLOT):
            qd = k // 2 + 1
            op = CYCLE[(pos - qd) % N_DEV]
            om = CYCLE[(pos + qd) % N_DEV]
            row_p.append(op * m_per + (0 if k % 2 == 0 else half))
            row_m.append(om * m_per + (half if k % 2 == 0 else 0))
        rows.append([d * m_per, left, right] + row_p + row_m)
    return np.asarray(rows, np.int32)


def _silu(y):
    return y * jax.nn.sigmoid(y)


def kernel(x, w_mat):
    m_per, kdim = x.shape
    _, n_per = w_mat.shape
    half = m_per // 2

    table = jnp.asarray(_meta_table(m_per))

    def body(tbl_ref, x_ref, w_ref, out_ref,
             bufp, bufm, send_p, recv_p, send_m, recv_m):
        my = lax.axis_index("i")
        meta_ref = tbl_ref.at[my]
        left_id = meta_ref[1]
        right_id = meta_ref[2]

        barrier_sem = pltpu.get_barrier_semaphore()
        for nbr in (left_id, right_id):
            pl.semaphore_signal(
                barrier_sem, inc=1,
                device_id=(nbr,), device_id_type=pl.DeviceIdType.MESH,
            )
        pl.semaphore_wait(barrier_sem, 2)

        def mk(buf, send_sems, recv_sems, slot, src_ref, dev):
            return pltpu.make_async_remote_copy(
                src_ref=src_ref,
                dst_ref=buf.at[slot],
                send_sem=send_sems.at[slot],
                recv_sem=recv_sems.at[slot],
                device_id=(dev,),
                device_id_type=pl.DeviceIdType.MESH,
            )

        x_a = x_ref.at[pl.ds(0, half), :]
        x_b = x_ref.at[pl.ds(half, half), :]
        descs_p = {
            0: mk(bufp, send_p, recv_p, 0, x_a, right_id),
            1: mk(bufp, send_p, recv_p, 1, x_b, right_id),
        }
        descs_m = {
            0: mk(bufm, send_m, recv_m, 0, x_b, left_id),
            1: mk(bufm, send_m, recv_m, 1, x_a, left_id),
        }
        descs_p[0].start()
        descs_m[0].start()
        descs_p[1].start()
        descs_m[1].start()

        y = jnp.dot(x_ref[:, :], w_ref[:, :],
                    preferred_element_type=jnp.float32)
        out_ref[pl.ds(meta_ref[0], m_per), :] = _silu(y)

        for h in range(NSLOT):
            descs_p[h].wait_recv()
            if h <= NSLOT - 3:
                d = mk(bufp, send_p, recv_p, h + 2, bufp.at[h], right_id)
                descs_p[h + 2] = d
                d.start()
            descs_m[h].wait_recv()
            if h <= NSLOT - 3:
                d = mk(bufm, send_m, recv_m, h + 2, bufm.at[h], left_id)
                descs_m[h + 2] = d
                d.start()
            yp = jnp.dot(bufp[h, :, :], w_ref[:, :],
                         preferred_element_type=jnp.float32)
            out_ref[pl.ds(meta_ref[3 + h], half), :] = _silu(yp)
            ym = jnp.dot(bufm[h, :, :], w_ref[:, :],
                         preferred_element_type=jnp.float32)
            out_ref[pl.ds(meta_ref[3 + NSLOT + h], half), :] = _silu(ym)

        for h in range(NSLOT):
            descs_p[h].wait_send()
            descs_m[h].wait_send()

    return pl.pallas_call(
        body,
        out_shape=jax.ShapeDtypeStruct((N_DEV * m_per, n_per), jnp.float32),
        in_specs=[
            pl.BlockSpec(memory_space=pltpu.SMEM),
            pl.BlockSpec(memory_space=pltpu.VMEM),
            pl.BlockSpec(memory_space=pltpu.VMEM),
        ],
        out_specs=pl.BlockSpec(memory_space=pltpu.VMEM),
        scratch_shapes=[
            pltpu.VMEM((NSLOT, half, kdim), jnp.float32),
            pltpu.VMEM((NSLOT, half, kdim), jnp.float32),
            pltpu.SemaphoreType.DMA((NSLOT,)),
            pltpu.SemaphoreType.DMA((NSLOT,)),
            pltpu.SemaphoreType.DMA((NSLOT,)),
            pltpu.SemaphoreType.DMA((NSLOT,)),
        ],
        compiler_params=pltpu.CompilerParams(collective_id=0),
    )(table, x, w_mat)
